# baseline (speedup 1.0000x reference)
import jax
import jax.numpy as jnp
from jax.experimental import pallas as pl
from jax.experimental.pallas import tpu as pltpu

_BM = 128


def _router_body(x_hbm, w1_ref, b1_ref, w2_ref, b2_ref, o_hbm):
    n, d = x_hbm.shape
    ne = o_hbm.shape[1]

    def _inner(x_ref, o_ref):
        x = x_ref[...].astype(jnp.bfloat16)
        h = jnp.dot(x, w1_ref[...], preferred_element_type=jnp.float32)
        h = jnp.maximum(h + b1_ref[...], 0.0)
        logits = jnp.dot(h, w2_ref[...], preferred_element_type=jnp.float32)
        logits = logits + b2_ref[...]
        m = jnp.max(logits, axis=1, keepdims=True)
        e = jnp.exp(logits - m)
        o_ref[...] = e / jnp.sum(e, axis=1, keepdims=True)

    pipe = pltpu.emit_pipeline(
        _inner,
        grid=(n // _BM,),
        in_specs=[
            pl.BlockSpec((_BM, d), lambda i: (i, 0),
                         pipeline_mode=pl.Buffered(buffer_count=16)),
        ],
        out_specs=[
            pl.BlockSpec((_BM, ne), lambda i: (i, 0)),
        ],
    )
    pipe(x_hbm, o_hbm)


def kernel(prnet_features, W1, b1, W2, b2):
    n, d = prnet_features.shape
    hidden = W1.shape[1]
    ne = W2.shape[1]
    return pl.pallas_call(
        _router_body,
        in_specs=[
            pl.BlockSpec(memory_space=pltpu.MemorySpace.HBM),
            pl.BlockSpec(memory_space=pltpu.MemorySpace.VMEM),
            pl.BlockSpec(memory_space=pltpu.MemorySpace.VMEM),
            pl.BlockSpec(memory_space=pltpu.MemorySpace.VMEM),
            pl.BlockSpec(memory_space=pltpu.MemorySpace.VMEM),
        ],
        out_specs=pl.BlockSpec(memory_space=pltpu.MemorySpace.HBM),
        out_shape=jax.ShapeDtypeStruct((n, ne), jnp.float32),
        compiler_params=pltpu.CompilerParams(
            vmem_limit_bytes=63 * 1024 * 1024,
        ),
    )(prnet_features, W1.astype(jnp.bfloat16), b1.reshape(1, hidden),
      W2, b2.reshape(1, ne))


# speedup vs baseline: 1.2213x; 1.2213x over previous
import jax
import jax.numpy as jnp
from jax.experimental import pallas as pl
from jax.experimental.pallas import tpu as pltpu

_BM = 256


def _router_body(x_hbm, w1_ref, b1_ref, w2_ref, b2_ref, o_hbm):
    n, d = x_hbm.shape
    ne = o_hbm.shape[1]

    def _inner(x_ref, o_ref):
        x = x_ref[...].astype(jnp.bfloat16)
        h = jnp.dot(x, w1_ref[...], preferred_element_type=jnp.float32)
        h = jnp.maximum(h + b1_ref[...], 0.0)
        logits = jnp.dot(h, w2_ref[...], preferred_element_type=jnp.float32)
        logits = logits + b2_ref[...]
        m = jnp.max(logits, axis=1, keepdims=True)
        e = jnp.exp(logits - m)
        o_ref[...] = e / jnp.sum(e, axis=1, keepdims=True)

    pipe = pltpu.emit_pipeline(
        _inner,
        grid=(n // _BM,),
        in_specs=[
            pl.BlockSpec((_BM, d), lambda i: (i, 0),
                         pipeline_mode=pl.Buffered(buffer_count=12)),
        ],
        out_specs=[
            pl.BlockSpec((_BM, ne), lambda i: (i, 0)),
        ],
    )
    pipe(x_hbm, o_hbm)


def kernel(prnet_features, W1, b1, W2, b2):
    n, d = prnet_features.shape
    hidden = W1.shape[1]
    ne = W2.shape[1]
    return pl.pallas_call(
        _router_body,
        in_specs=[
            pl.BlockSpec(memory_space=pltpu.MemorySpace.HBM),
            pl.BlockSpec(memory_space=pltpu.MemorySpace.VMEM),
            pl.BlockSpec(memory_space=pltpu.MemorySpace.VMEM),
            pl.BlockSpec(memory_space=pltpu.MemorySpace.VMEM),
            pl.BlockSpec(memory_space=pltpu.MemorySpace.VMEM),
        ],
        out_specs=pl.BlockSpec(memory_space=pltpu.MemorySpace.HBM),
        out_shape=jax.ShapeDtypeStruct((n, ne), jnp.float32),
        compiler_params=pltpu.CompilerParams(
            vmem_limit_bytes=63 * 1024 * 1024,
        ),
    )(prnet_features, W1.astype(jnp.bfloat16), b1.reshape(1, hidden),
      W2, b2.reshape(1, ne))


# speedup vs baseline: 1.2346x; 1.0109x over previous
import jax
import jax.numpy as jnp
from jax.experimental import pallas as pl
from jax.experimental.pallas import tpu as pltpu

_BM = 256


def _router_body(x_hbm, w1_ref, b1_ref, w2_ref, b2_ref, o_hbm):
    n, d = x_hbm.shape
    ne = o_hbm.shape[1]

    def _inner(x_ref, o_ref):
        x = x_ref[...]
        h = jnp.dot(x, w1_ref[...], preferred_element_type=jnp.float32)
        h = jnp.maximum(h + b1_ref[...], 0.0)
        logits = jnp.dot(h, w2_ref[...], preferred_element_type=jnp.float32)
        logits = logits + b2_ref[...]
        m = jnp.max(logits, axis=1, keepdims=True)
        e = jnp.exp(logits - m)
        o_ref[...] = e / jnp.sum(e, axis=1, keepdims=True)

    pipe = pltpu.emit_pipeline(
        _inner,
        grid=(n // _BM,),
        in_specs=[
            pl.BlockSpec((_BM, d), lambda i: (i, 0),
                         pipeline_mode=pl.Buffered(buffer_count=8)),
        ],
        out_specs=[
            pl.BlockSpec((_BM, ne), lambda i: (i, 0)),
        ],
    )
    pipe(x_hbm, o_hbm)


def kernel(prnet_features, W1, b1, W2, b2):
    n, d = prnet_features.shape
    hidden = W1.shape[1]
    ne = W2.shape[1]
    return pl.pallas_call(
        _router_body,
        in_specs=[
            pl.BlockSpec(memory_space=pltpu.MemorySpace.HBM),
            pl.BlockSpec(memory_space=pltpu.MemorySpace.VMEM),
            pl.BlockSpec(memory_space=pltpu.MemorySpace.VMEM),
            pl.BlockSpec(memory_space=pltpu.MemorySpace.VMEM),
            pl.BlockSpec(memory_space=pltpu.MemorySpace.VMEM),
        ],
        out_specs=pl.BlockSpec(memory_space=pltpu.MemorySpace.HBM),
        out_shape=jax.ShapeDtypeStruct((n, ne), jnp.float32),
        compiler_params=pltpu.CompilerParams(
            vmem_limit_bytes=63 * 1024 * 1024,
        ),
    )(prnet_features, W1, b1.reshape(1, hidden),
      W2, b2.reshape(1, ne))
